# baseline (speedup 1.0000x reference)
import functools

import jax
import jax.numpy as jnp
import numpy as np
from jax import lax
from jax.experimental import pallas as pl
from jax.experimental.pallas import tpu as pltpu
from jax.experimental.pallas import tpu_sc as plsc

N_NODES = 10000
D = 128
DH = D // 2
LANES = 16
N_CORES = 2
N_SUBCORES = 16
CHUNK = 128
CHUNKS_PER_W = 160
ROUND = 40
N_ROUNDS = CHUNKS_PER_W // ROUND
N_CHUNKS = N_SUBCORES * CHUNKS_PER_W
E_PAD = N_CHUNKS * CHUNK
ROWS_PER_TILE = 624

_COL_PERM = np.empty(DH, np.int32)
for _g in range(DH // 32):
    for _t in range(16):
        _COL_PERM[32 * _g + 2 * _t] = 32 * _g + _t
        _COL_PERM[32 * _g + 2 * _t + 1] = 32 * _g + 16 + _t


def _mm_halves_kernel(x_ref, w_ref, o_ref):
    o_ref[...] = jnp.dot(x_ref[...], w_ref[0],
                         preferred_element_type=jnp.float32
                         ).astype(jnp.bfloat16)


def _tc_matmul_halves(x, w3):
    m = x.shape[0]
    bm = 1000
    return pl.pallas_call(
        _mm_halves_kernel,
        grid=(2, m // bm),
        in_specs=[pl.BlockSpec((bm, D), lambda c, i: (i, 0)),
                  pl.BlockSpec((1, D, DH), lambda c, i: (c, 0, 0))],
        out_specs=pl.BlockSpec((bm, DH), lambda c, i: (c * (m // bm) + i, 0)),
        out_shape=jax.ShapeDtypeStruct((2 * m, DH), jnp.bfloat16),
    )(x, w3)


def _combine_relu_mm_kernel(pa_ref, pb_ref, b_ref, w_ref, o_ref):
    z = jnp.concatenate([pa_ref[...], pb_ref[...]], axis=1)
    z = jnp.maximum(z + b_ref[...], 0.0)
    o_ref[...] = jnp.dot(z, w_ref[0], preferred_element_type=jnp.float32
                         ).astype(jnp.bfloat16)


def _tc_combine_relu_matmul_halves(pa, pb, b, w3):
    m = pa.shape[0]
    bm = 1000
    return pl.pallas_call(
        _combine_relu_mm_kernel,
        grid=(2, m // bm),
        in_specs=[pl.BlockSpec((bm, DH), lambda c, i: (i, 0)),
                  pl.BlockSpec((bm, DH), lambda c, i: (i, 0)),
                  pl.BlockSpec((1, D), lambda c, i: (0, 0)),
                  pl.BlockSpec((1, D, DH), lambda c, i: (c, 0, 0))],
        out_specs=pl.BlockSpec((bm, DH), lambda c, i: (c * (m // bm) + i, 0)),
        out_shape=jax.ShapeDtypeStruct((2 * m, DH), jnp.bfloat16),
    )(pa, pb, b, w3)


def _combine_bias_kernel(qa_ref, qb_ref, b_ref, o_ref):
    o_ref[...] = jnp.concatenate([qa_ref[...], qb_ref[...]], axis=1) + b_ref[...]


def _tc_combine_bias(qa, qb, b):
    m = qa.shape[0]
    bm = 1000
    return pl.pallas_call(
        _combine_bias_kernel,
        grid=(m // bm,),
        in_specs=[pl.BlockSpec((bm, DH), lambda i: (i, 0)),
                  pl.BlockSpec((bm, DH), lambda i: (i, 0)),
                  pl.BlockSpec((1, D), lambda i: (0, 0))],
        out_specs=pl.BlockSpec((bm, D), lambda i: (i, 0)),
        out_shape=jax.ShapeDtypeStruct((m, D), jnp.float32),
    )(qa, qb, b)


def _sc_spmm(h2, src3d, dst2d, ew2d):
    mesh = plsc.VectorSubcoreMesh(core_axis_name="c", subcore_axis_name="s")

    @functools.partial(
        pl.kernel,
        out_type=jax.ShapeDtypeStruct((N_CORES, N_NODES, DH), jnp.float32),
        mesh=mesh,
        compiler_params=pltpu.CompilerParams(use_tc_tiling_on_sc=False,
                                             needs_layout_passes=False),
        scratch_types=[
            pltpu.VMEM((ROUND, CHUNK), jnp.int32),
            pltpu.VMEM((ROUND, CHUNK), jnp.int32),
            pltpu.VMEM((ROUND, CHUNK), jnp.float32),
            [pltpu.VMEM((CHUNK, DH // 2), jnp.int32) for _ in range(2)],
            [pltpu.VMEM((CHUNK, DH), jnp.float32) for _ in range(2)],
            pltpu.VMEM_SHARED((N_NODES, DH), jnp.float32),
            [pltpu.SemaphoreType.DMA for _ in range(2)],
            [pltpu.SemaphoreType.DMA for _ in range(2)],
        ],
    )
    def spmm(h_hbm, src_hbm, dst_hbm, ew_hbm, out_hbm,
             src_v, dst_v, ew_v, rbufs, sbufs, agg_sh, gsems, ssems):
        cid = lax.axis_index("c")
        sid = lax.axis_index("s")

        zb = sbufs[0]

        @pl.loop(0, CHUNK)
        def _(i):
            for k in range(DH // LANES):
                zb[i, pl.ds(k * LANES, LANES)] = jnp.zeros((LANES,),
                                                           jnp.float32)

        def zero_span(start, total):
            full, rem = divmod(total, CHUNK)
            for k in range(full):
                pltpu.sync_copy(zb.at[pl.ds(0, CHUNK)],
                                agg_sh.at[pl.ds(start + k * CHUNK, CHUNK)])
            if rem:
                pltpu.sync_copy(zb.at[pl.ds(0, rem)],
                                agg_sh.at[pl.ds(start + full * CHUNK, rem)])

        @pl.when(sid < N_SUBCORES - 1)
        def _():
            zero_span(sid * ROWS_PER_TILE, ROWS_PER_TILE)

        @pl.when(sid == N_SUBCORES - 1)
        def _():
            zero_span((N_SUBCORES - 1) * ROWS_PER_TILE,
                      N_NODES - (N_SUBCORES - 1) * ROWS_PER_TILE)
        plsc.subcore_barrier()

        def start_gather(c, buf, sem):
            pltpu.async_copy(h_hbm.at[src_v.at[c]], buf, sem)

        def wait_gather(c, buf, sem):
            pltpu.make_async_copy(h_hbm.at[src_v.at[c]], buf, sem).wait()

        def start_scatter(c, buf, sem):
            pltpu.async_copy(buf, agg_sh.at[dst_v.at[c]], sem, add=True)

        def wait_scatter(c, buf, sem):
            pltpu.make_async_copy(buf, agg_sh.at[dst_v.at[c]], sem).wait()

        def scale(c, src_buf, dst_buf):
            @pl.loop(0, CHUNK // LANES)
            def _(g):
                ewv = ew_v[c, pl.ds(g * LANES, LANES)]
                for l in range(LANES):
                    wgt = ewv[l]
                    i = g * LANES + l
                    for k in range(DH // 32):
                        w = src_buf[i, pl.ds(k * LANES, LANES)]
                        lo = plsc.bitcast(w << 16, jnp.float32)
                        hi = plsc.bitcast(w & np.int32(-65536), jnp.float32)
                        dst_buf[i, pl.ds(32 * k, LANES)] = lo * wgt
                        dst_buf[i, pl.ds(32 * k + LANES, LANES)] = hi * wgt

        @pl.loop(0, N_ROUNDS)
        def _(r):
            row0 = sid * CHUNKS_PER_W + r * ROUND
            pltpu.sync_copy(src_hbm.at[cid, pl.ds(row0, ROUND)], src_v)
            pltpu.sync_copy(dst_hbm.at[pl.ds(row0, ROUND)], dst_v)
            pltpu.sync_copy(ew_hbm.at[pl.ds(row0, ROUND)], ew_v)

            start_gather(0, rbufs[0], gsems[0])
            start_gather(1, rbufs[1], gsems[1])

            @pl.loop(0, ROUND, step=2)
            def _(c0):
                for u in range(2):
                    c = c0 + u
                    wait_gather(c, rbufs[u], gsems[u])

                    @pl.when(c >= 2)
                    def _():
                        wait_scatter(c - 2, sbufs[u], ssems[u])
                    scale(c, rbufs[u], sbufs[u])
                    start_scatter(c, sbufs[u], ssems[u])

                    @pl.when(c + 2 < ROUND)
                    def _():
                        start_gather(c + 2, rbufs[u], gsems[u])

            for u in range(2):
                c = ROUND - 2 + u
                wait_scatter(c, sbufs[u], ssems[u])

        plsc.subcore_barrier()

        @pl.when(sid < N_SUBCORES - 1)
        def _():
            pltpu.sync_copy(
                agg_sh.at[pl.ds(sid * ROWS_PER_TILE, ROWS_PER_TILE)],
                out_hbm.at[cid, pl.ds(sid * ROWS_PER_TILE, ROWS_PER_TILE)])

        @pl.when(sid == N_SUBCORES - 1)
        def _():
            last0 = (N_SUBCORES - 1) * ROWS_PER_TILE
            pltpu.sync_copy(
                agg_sh.at[pl.ds(last0, N_NODES - last0)],
                out_hbm.at[cid, pl.ds(last0, N_NODES - last0)])

    return spmm(h2, src3d, dst2d, ew2d)


def kernel(x, edge_index, edge_weight, W0, b0, W1, b1):
    e = edge_index.shape[1]
    src = edge_index[0].astype(jnp.int32)
    dst = edge_index[1].astype(jnp.int32)
    ew = edge_weight.astype(jnp.float32)
    pad = E_PAD - e
    src2d = jnp.pad(src, (0, pad)).reshape(N_CHUNKS, CHUNK)
    dst2d = jnp.pad(dst, (0, pad)).reshape(N_CHUNKS, CHUNK)
    ew2d = jnp.pad(ew, (0, pad)).reshape(N_CHUNKS, CHUNK)
    src3d = jnp.stack([src2d, src2d + N_NODES])
    b0r = b0.reshape(1, D)
    b1r = b1.reshape(1, D)
    perm = jnp.asarray(_COL_PERM)
    w0s = jnp.stack([W0[:, :DH][:, perm], W0[:, DH:][:, perm]])
    w1s = jnp.stack([W1[:, :DH][:, perm], W1[:, DH:][:, perm]])

    def pack_i32(t):
        return jax.lax.bitcast_convert_type(
            t.reshape(t.shape[0], DH // 2, 2), jnp.int32)

    h0 = pack_i32(_tc_matmul_halves(x, w0s))
    p = _sc_spmm(h0, src3d, dst2d, ew2d)
    h1 = pack_i32(_tc_combine_relu_matmul_halves(p[0], p[1], b0r, w1s))
    q = _sc_spmm(h1, src3d, dst2d, ew2d)
    return _tc_combine_bias(q[0], q[1], b1r)


# speedup vs baseline: 1.2842x; 1.2842x over previous
import functools

import jax
import jax.numpy as jnp
import numpy as np
from jax import lax
from jax.experimental import pallas as pl
from jax.experimental.pallas import tpu as pltpu
from jax.experimental.pallas import tpu_sc as plsc

N_NODES = 10000
D = 128
DH = D // 2
LANES = 16
N_CORES = 2
N_SUBCORES = 16
CHUNK = 128
CHUNKS_PER_W = 160
ROUND = 40
N_ROUNDS = CHUNKS_PER_W // ROUND
N_CHUNKS = N_SUBCORES * CHUNKS_PER_W
E_PAD = N_CHUNKS * CHUNK
ROWS_PER_TILE = 624

_COL_PERM = np.empty(DH, np.int32)
for _g in range(DH // 32):
    for _t in range(16):
        _COL_PERM[32 * _g + 2 * _t] = 32 * _g + _t
        _COL_PERM[32 * _g + 2 * _t + 1] = 32 * _g + 16 + _t


def _mm_halves_kernel(x_ref, w_ref, o_ref):
    o_ref[...] = jnp.dot(x_ref[...], w_ref[0],
                         preferred_element_type=jnp.float32
                         ).astype(jnp.bfloat16)


def _tc_matmul_halves(x, w3):
    m = x.shape[0]
    bm = 1000
    return pl.pallas_call(
        _mm_halves_kernel,
        grid=(2, m // bm),
        in_specs=[pl.BlockSpec((bm, D), lambda c, i: (i, 0)),
                  pl.BlockSpec((1, D, DH), lambda c, i: (c, 0, 0))],
        out_specs=pl.BlockSpec((bm, DH), lambda c, i: (c * (m // bm) + i, 0)),
        out_shape=jax.ShapeDtypeStruct((2 * m, DH), jnp.bfloat16),
    )(x, w3)


def _combine_relu_mm_kernel(pa_ref, pb_ref, b_ref, w_ref, o_ref):
    z = jnp.concatenate([pa_ref[...], pb_ref[...]], axis=1)
    z = jnp.maximum(z + b_ref[...], 0.0)
    o_ref[...] = jnp.dot(z, w_ref[0], preferred_element_type=jnp.float32
                         ).astype(jnp.bfloat16)


def _tc_combine_relu_matmul_halves(pa, pb, b, w3):
    m = pa.shape[0]
    bm = 1000
    return pl.pallas_call(
        _combine_relu_mm_kernel,
        grid=(2, m // bm),
        in_specs=[pl.BlockSpec((bm, DH), lambda c, i: (i, 0)),
                  pl.BlockSpec((bm, DH), lambda c, i: (i, 0)),
                  pl.BlockSpec((1, D), lambda c, i: (0, 0)),
                  pl.BlockSpec((1, D, DH), lambda c, i: (c, 0, 0))],
        out_specs=pl.BlockSpec((bm, DH), lambda c, i: (c * (m // bm) + i, 0)),
        out_shape=jax.ShapeDtypeStruct((2 * m, DH), jnp.bfloat16),
    )(pa, pb, b, w3)


def _combine_bias_kernel(qa_ref, qb_ref, b_ref, o_ref):
    o_ref[...] = jnp.concatenate([qa_ref[...], qb_ref[...]], axis=1) + b_ref[...]


def _tc_combine_bias(qa, qb, b):
    m = qa.shape[0]
    bm = 1000
    return pl.pallas_call(
        _combine_bias_kernel,
        grid=(m // bm,),
        in_specs=[pl.BlockSpec((bm, DH), lambda i: (i, 0)),
                  pl.BlockSpec((bm, DH), lambda i: (i, 0)),
                  pl.BlockSpec((1, D), lambda i: (0, 0))],
        out_specs=pl.BlockSpec((bm, D), lambda i: (i, 0)),
        out_shape=jax.ShapeDtypeStruct((m, D), jnp.float32),
    )(qa, qb, b)


def _sc_spmm(h2, src3d, dst2d, ew2d):
    mesh = plsc.VectorSubcoreMesh(core_axis_name="c", subcore_axis_name="s")

    @functools.partial(
        pl.kernel,
        out_type=jax.ShapeDtypeStruct((N_CORES, N_NODES, DH), jnp.float32),
        mesh=mesh,
        compiler_params=pltpu.CompilerParams(use_tc_tiling_on_sc=False,
                                             needs_layout_passes=False),
        scratch_types=[
            pltpu.VMEM((ROUND, CHUNK), jnp.int32),
            pltpu.VMEM((ROUND, CHUNK), jnp.int32),
            pltpu.VMEM((ROUND, CHUNK), jnp.float32),
            [pltpu.VMEM((CHUNK, DH // 2), jnp.int32) for _ in range(2)],
            [pltpu.VMEM((CHUNK, DH), jnp.float32) for _ in range(2)],
            pltpu.VMEM_SHARED((N_NODES, DH), jnp.float32),
            [pltpu.SemaphoreType.DMA for _ in range(2)],
            [pltpu.SemaphoreType.DMA for _ in range(2)],
        ],
    )
    def spmm(h_hbm, src_hbm, dst_hbm, ew_hbm, out_hbm,
             src_v, dst_v, ew_v, rbufs, sbufs, agg_sh, gsems, ssems):
        cid = lax.axis_index("c")
        sid = lax.axis_index("s")

        zb = sbufs[0]

        @pl.loop(0, CHUNK)
        def _(i):
            for k in range(DH // LANES):
                zb[i, pl.ds(k * LANES, LANES)] = jnp.zeros((LANES,),
                                                           jnp.float32)

        def zero_span(start, total):
            full, rem = divmod(total, CHUNK)
            for k in range(full):
                pltpu.sync_copy(zb.at[pl.ds(0, CHUNK)],
                                agg_sh.at[pl.ds(start + k * CHUNK, CHUNK)])
            if rem:
                pltpu.sync_copy(zb.at[pl.ds(0, rem)],
                                agg_sh.at[pl.ds(start + full * CHUNK, rem)])

        @pl.when(sid < N_SUBCORES - 1)
        def _():
            zero_span(sid * ROWS_PER_TILE, ROWS_PER_TILE)

        @pl.when(sid == N_SUBCORES - 1)
        def _():
            zero_span((N_SUBCORES - 1) * ROWS_PER_TILE,
                      N_NODES - (N_SUBCORES - 1) * ROWS_PER_TILE)
        plsc.subcore_barrier()

        def start_gather(c, buf, sem):
            pltpu.async_copy(h_hbm.at[src_v.at[c]], buf, sem)

        def wait_gather(c, buf, sem):
            pltpu.make_async_copy(h_hbm.at[src_v.at[c]], buf, sem).wait()

        def start_scatter(c, buf, sem):
            pltpu.async_copy(buf, agg_sh.at[dst_v.at[c]], sem, add=True)

        def wait_scatter(c, buf, sem):
            pltpu.make_async_copy(buf, agg_sh.at[dst_v.at[c]], sem).wait()

        def scale(c, src_buf, dst_buf):
            NB = 4
            NG = DH // 32

            @pl.loop(0, CHUNK // LANES)
            def _(g):
                ewv = ew_v[c, pl.ds(g * LANES, LANES)]
                for l0 in range(0, LANES, NB):
                    rows = [g * LANES + l0 + j for j in range(NB)]
                    wgts = [ewv[l0 + j] for j in range(NB)]
                    ws = [[src_buf[i, pl.ds(k * LANES, LANES)]
                           for k in range(NG)] for i in rows]
                    for j in range(NB):
                        for k in range(NG):
                            w = ws[j][k]
                            lo = plsc.bitcast(w << 16, jnp.float32)
                            hi = plsc.bitcast(w & np.int32(-65536),
                                              jnp.float32)
                            dst_buf[rows[j], pl.ds(32 * k, LANES)] = (
                                lo * wgts[j])
                            dst_buf[rows[j], pl.ds(32 * k + LANES, LANES)] = (
                                hi * wgts[j])

        @pl.loop(0, N_ROUNDS)
        def _(r):
            row0 = sid * CHUNKS_PER_W + r * ROUND
            pltpu.sync_copy(src_hbm.at[cid, pl.ds(row0, ROUND)], src_v)
            pltpu.sync_copy(dst_hbm.at[pl.ds(row0, ROUND)], dst_v)
            pltpu.sync_copy(ew_hbm.at[pl.ds(row0, ROUND)], ew_v)

            start_gather(0, rbufs[0], gsems[0])
            start_gather(1, rbufs[1], gsems[1])

            @pl.loop(0, ROUND, step=2)
            def _(c0):
                for u in range(2):
                    c = c0 + u
                    wait_gather(c, rbufs[u], gsems[u])

                    @pl.when(c >= 2)
                    def _():
                        wait_scatter(c - 2, sbufs[u], ssems[u])
                    scale(c, rbufs[u], sbufs[u])
                    start_scatter(c, sbufs[u], ssems[u])

                    @pl.when(c + 2 < ROUND)
                    def _():
                        start_gather(c + 2, rbufs[u], gsems[u])

            for u in range(2):
                c = ROUND - 2 + u
                wait_scatter(c, sbufs[u], ssems[u])

        plsc.subcore_barrier()

        @pl.when(sid < N_SUBCORES - 1)
        def _():
            pltpu.sync_copy(
                agg_sh.at[pl.ds(sid * ROWS_PER_TILE, ROWS_PER_TILE)],
                out_hbm.at[cid, pl.ds(sid * ROWS_PER_TILE, ROWS_PER_TILE)])

        @pl.when(sid == N_SUBCORES - 1)
        def _():
            last0 = (N_SUBCORES - 1) * ROWS_PER_TILE
            pltpu.sync_copy(
                agg_sh.at[pl.ds(last0, N_NODES - last0)],
                out_hbm.at[cid, pl.ds(last0, N_NODES - last0)])

    return spmm(h2, src3d, dst2d, ew2d)


def kernel(x, edge_index, edge_weight, W0, b0, W1, b1):
    e = edge_index.shape[1]
    src = edge_index[0].astype(jnp.int32)
    dst = edge_index[1].astype(jnp.int32)
    ew = edge_weight.astype(jnp.float32)
    pad = E_PAD - e
    src2d = jnp.pad(src, (0, pad)).reshape(N_CHUNKS, CHUNK)
    dst2d = jnp.pad(dst, (0, pad)).reshape(N_CHUNKS, CHUNK)
    ew2d = jnp.pad(ew, (0, pad)).reshape(N_CHUNKS, CHUNK)
    src3d = jnp.stack([src2d, src2d + N_NODES])
    b0r = b0.reshape(1, D)
    b1r = b1.reshape(1, D)
    perm = jnp.asarray(_COL_PERM)
    w0s = jnp.stack([W0[:, :DH][:, perm], W0[:, DH:][:, perm]])
    w1s = jnp.stack([W1[:, :DH][:, perm], W1[:, DH:][:, perm]])

    def pack_i32(t):
        return jax.lax.bitcast_convert_type(
            t.reshape(t.shape[0], DH // 2, 2), jnp.int32)

    h0 = pack_i32(_tc_matmul_halves(x, w0s))
    p = _sc_spmm(h0, src3d, dst2d, ew2d)
    h1 = pack_i32(_tc_combine_relu_matmul_halves(p[0], p[1], b0r, w1s))
    q = _sc_spmm(h1, src3d, dst2d, ew2d)
    return _tc_combine_bias(q[0], q[1], b1r)


# speedup vs baseline: 1.7192x; 1.3387x over previous
import functools

import jax
import jax.numpy as jnp
import numpy as np
from jax import lax
from jax.experimental import pallas as pl
from jax.experimental.pallas import tpu as pltpu
from jax.experimental.pallas import tpu_sc as plsc

N_NODES = 10000
D = 128
DH = D // 2
LANES = 16
BLANES = 2 * LANES
N_CORES = 2
N_SUBCORES = 16
CHUNK = 128
CHUNKS_PER_W = 160
ROUND = 40
N_ROUNDS = CHUNKS_PER_W // ROUND
N_CHUNKS = N_SUBCORES * CHUNKS_PER_W
E_PAD = N_CHUNKS * CHUNK
ROWS_PER_TILE = 624
BM = 2000


def _mm_halves_kernel(x_ref, w_ref, o_ref):
    o_ref[...] = jnp.dot(x_ref[...], w_ref[0],
                         preferred_element_type=jnp.float32
                         ).astype(jnp.bfloat16)


def _tc_matmul_halves(x, w3):
    m = x.shape[0]
    return pl.pallas_call(
        _mm_halves_kernel,
        grid=(2, m // BM),
        in_specs=[pl.BlockSpec((BM, D), lambda c, i: (i, 0)),
                  pl.BlockSpec((1, D, DH), lambda c, i: (c, 0, 0))],
        out_specs=pl.BlockSpec((BM, DH), lambda c, i: (c * (m // BM) + i, 0)),
        out_shape=jax.ShapeDtypeStruct((2 * m, DH), jnp.bfloat16),
    )(x, w3)


def _combine_relu_mm_kernel(pa_ref, pb_ref, b_ref, w_ref, o_ref):
    z = jnp.concatenate([pa_ref[...].astype(jnp.float32),
                         pb_ref[...].astype(jnp.float32)], axis=1)
    z = jnp.maximum(z + b_ref[...], 0.0)
    o_ref[...] = jnp.dot(z, w_ref[0], preferred_element_type=jnp.float32
                         ).astype(jnp.bfloat16)


def _tc_combine_relu_matmul_halves(pa, pb, b, w3):
    m = pa.shape[0]
    return pl.pallas_call(
        _combine_relu_mm_kernel,
        grid=(2, m // BM),
        in_specs=[pl.BlockSpec((BM, DH), lambda c, i: (i, 0)),
                  pl.BlockSpec((BM, DH), lambda c, i: (i, 0)),
                  pl.BlockSpec((1, D), lambda c, i: (0, 0)),
                  pl.BlockSpec((1, D, DH), lambda c, i: (c, 0, 0))],
        out_specs=pl.BlockSpec((BM, DH), lambda c, i: (c * (m // BM) + i, 0)),
        out_shape=jax.ShapeDtypeStruct((2 * m, DH), jnp.bfloat16),
    )(pa, pb, b, w3)


def _combine_bias_kernel(qa_ref, qb_ref, b_ref, o_ref):
    o_ref[...] = jnp.concatenate(
        [qa_ref[...].astype(jnp.float32), qb_ref[...].astype(jnp.float32)],
        axis=1) + b_ref[...]


def _tc_combine_bias(qa, qb, b):
    m = qa.shape[0]
    return pl.pallas_call(
        _combine_bias_kernel,
        grid=(m // BM,),
        in_specs=[pl.BlockSpec((BM, DH), lambda i: (i, 0)),
                  pl.BlockSpec((BM, DH), lambda i: (i, 0)),
                  pl.BlockSpec((1, D), lambda i: (0, 0))],
        out_specs=pl.BlockSpec((BM, D), lambda i: (i, 0)),
        out_shape=jax.ShapeDtypeStruct((m, D), jnp.float32),
    )(qa, qb, b)


def _sc_spmm(h2, src3d, dst2d, ew2d):
    mesh = plsc.VectorSubcoreMesh(core_axis_name="c", subcore_axis_name="s")

    @functools.partial(
        pl.kernel,
        out_type=jax.ShapeDtypeStruct((N_CORES, N_NODES, DH), jnp.bfloat16),
        mesh=mesh,
        compiler_params=pltpu.CompilerParams(use_tc_tiling_on_sc=False,
                                             needs_layout_passes=False),
        scratch_types=[
            pltpu.VMEM((ROUND, CHUNK), jnp.int32),
            pltpu.VMEM((ROUND, CHUNK), jnp.int32),
            pltpu.VMEM((ROUND, CHUNK), jnp.int32),
            [pltpu.VMEM((CHUNK, DH), jnp.bfloat16) for _ in range(2)],
            [pltpu.VMEM((CHUNK, DH), jnp.bfloat16) for _ in range(2)],
            pltpu.VMEM_SHARED((N_NODES, DH), jnp.bfloat16),
            [pltpu.SemaphoreType.DMA for _ in range(2)],
            [pltpu.SemaphoreType.DMA for _ in range(2)],
        ],
    )
    def spmm(h_hbm, src_hbm, dst_hbm, ew_hbm, out_hbm,
             src_v, dst_v, ew_v, rbufs, sbufs, agg_sh, gsems, ssems):
        cid = lax.axis_index("c")
        sid = lax.axis_index("s")

        zb = sbufs[0]

        @pl.loop(0, CHUNK)
        def _(i):
            for k in range(DH // BLANES):
                zb[i, pl.ds(k * BLANES, BLANES)] = jnp.zeros(
                    (BLANES,), jnp.bfloat16)

        def zero_span(start, total):
            full, rem = divmod(total, CHUNK)
            for k in range(full):
                pltpu.sync_copy(zb.at[pl.ds(0, CHUNK)],
                                agg_sh.at[pl.ds(start + k * CHUNK, CHUNK)])
            if rem:
                pltpu.sync_copy(zb.at[pl.ds(0, rem)],
                                agg_sh.at[pl.ds(start + full * CHUNK, rem)])

        @pl.when(sid < N_SUBCORES - 1)
        def _():
            zero_span(sid * ROWS_PER_TILE, ROWS_PER_TILE)

        @pl.when(sid == N_SUBCORES - 1)
        def _():
            zero_span((N_SUBCORES - 1) * ROWS_PER_TILE,
                      N_NODES - (N_SUBCORES - 1) * ROWS_PER_TILE)
        plsc.subcore_barrier()

        def start_gather(c, buf, sem):
            pltpu.async_copy(h_hbm.at[src_v.at[c]], buf, sem)

        def wait_gather(c, buf, sem):
            pltpu.make_async_copy(h_hbm.at[src_v.at[c]], buf, sem).wait()

        def start_scatter(c, buf, sem):
            pltpu.async_copy(buf, agg_sh.at[dst_v.at[c]], sem, add=True)

        def wait_scatter(c, buf, sem):
            pltpu.make_async_copy(buf, agg_sh.at[dst_v.at[c]], sem).wait()

        def scale(c, src_buf, dst_buf):
            NB = 4
            NG = DH // BLANES

            @pl.loop(0, CHUNK // LANES)
            def _(g):
                ewv = ew_v[c, pl.ds(g * LANES, LANES)]
                for l0 in range(0, LANES, NB):
                    rows = [g * LANES + l0 + j for j in range(NB)]
                    wgts = [plsc.bitcast(
                        lax.broadcast_in_dim(ewv[l0 + j], (LANES,), ()),
                        jnp.bfloat16) for j in range(NB)]
                    vs = [[src_buf[i, pl.ds(k * BLANES, BLANES)]
                           for k in range(NG)] for i in rows]
                    for j in range(NB):
                        for k in range(NG):
                            dst_buf[rows[j], pl.ds(k * BLANES, BLANES)] = (
                                vs[j][k] * wgts[j])

        @pl.loop(0, N_ROUNDS)
        def _(r):
            row0 = sid * CHUNKS_PER_W + r * ROUND
            pltpu.sync_copy(src_hbm.at[cid, pl.ds(row0, ROUND)], src_v)
            pltpu.sync_copy(dst_hbm.at[pl.ds(row0, ROUND)], dst_v)
            pltpu.sync_copy(ew_hbm.at[pl.ds(row0, ROUND)], ew_v)

            start_gather(0, rbufs[0], gsems[0])
            start_gather(1, rbufs[1], gsems[1])

            @pl.loop(0, ROUND, step=2)
            def _(c0):
                for u in range(2):
                    c = c0 + u
                    wait_gather(c, rbufs[u], gsems[u])

                    @pl.when(c >= 2)
                    def _():
                        wait_scatter(c - 2, sbufs[u], ssems[u])
                    scale(c, rbufs[u], sbufs[u])
                    start_scatter(c, sbufs[u], ssems[u])

                    @pl.when(c + 2 < ROUND)
                    def _():
                        start_gather(c + 2, rbufs[u], gsems[u])

            for u in range(2):
                c = ROUND - 2 + u
                wait_scatter(c, sbufs[u], ssems[u])

        plsc.subcore_barrier()

        @pl.when(sid < N_SUBCORES - 1)
        def _():
            pltpu.sync_copy(
                agg_sh.at[pl.ds(sid * ROWS_PER_TILE, ROWS_PER_TILE)],
                out_hbm.at[cid, pl.ds(sid * ROWS_PER_TILE, ROWS_PER_TILE)])

        @pl.when(sid == N_SUBCORES - 1)
        def _():
            last0 = (N_SUBCORES - 1) * ROWS_PER_TILE
            pltpu.sync_copy(
                agg_sh.at[pl.ds(last0, N_NODES - last0)],
                out_hbm.at[cid, pl.ds(last0, N_NODES - last0)])

    return spmm(h2, src3d, dst2d, ew2d)


def kernel(x, edge_index, edge_weight, W0, b0, W1, b1):
    e = edge_index.shape[1]
    src = edge_index[0].astype(jnp.int32)
    dst = edge_index[1].astype(jnp.int32)
    ew = edge_weight.astype(jnp.float32)
    pad = E_PAD - e
    src2d = jnp.pad(src, (0, pad)).reshape(N_CHUNKS, CHUNK)
    dst2d = jnp.pad(dst, (0, pad)).reshape(N_CHUNKS, CHUNK)
    ewbits = jax.lax.bitcast_convert_type(
        ew.astype(jnp.bfloat16), jnp.uint16).astype(jnp.uint32)
    eww = jax.lax.bitcast_convert_type(ewbits | (ewbits << 16), jnp.int32)
    ew2d = jnp.pad(eww, (0, pad)).reshape(N_CHUNKS, CHUNK)
    src3d = jnp.stack([src2d, src2d + N_NODES])
    b0r = b0.reshape(1, D)
    b1r = b1.reshape(1, D)
    w0s = jnp.stack([W0[:, :DH], W0[:, DH:]])
    w1s = jnp.stack([W1[:, :DH], W1[:, DH:]])

    h0 = _tc_matmul_halves(x, w0s)
    p = _sc_spmm(h0, src3d, dst2d, ew2d)
    h1 = _tc_combine_relu_matmul_halves(p[0], p[1], b0r, w1s)
    q = _sc_spmm(h1, src3d, dst2d, ew2d)
    return _tc_combine_bias(q[0], q[1], b1r)
